# Initial kernel scaffold; baseline (speedup 1.0000x reference)
import jax
import jax.numpy as jnp
from jax.experimental import pallas as pl


def kernel(position, embedding):
    raise NotImplementedError("write your pallas kernel here")


# speedup vs baseline: 5.5265x; 5.5265x over previous
import functools

import jax
import jax.numpy as jnp
from jax import lax
from jax.experimental import pallas as pl
from jax.experimental.pallas import tpu as pltpu
from jax.experimental.pallas import tpu_sc as plsc

EMBED = 64
PERIOD = 30
LANES = 16
CHUNK = 512
PAIRS = CHUNK // 2
IDX_ROWS = PAIRS // 128


def _sc_lookup(pos_flat, ptable, num_workers):
    n = pos_flat.shape[0]
    b_per_w = n // num_workers
    n_chunks = b_per_w // CHUNK

    mesh = plsc.VectorSubcoreMesh(core_axis_name="c", subcore_axis_name="s")

    @functools.partial(
        pl.kernel,
        out_type=jax.ShapeDtypeStruct((n // 2, 2 * EMBED), jnp.float32),
        mesh=mesh,
        scratch_types=[
            pltpu.VMEM((CHUNK,), jnp.int32),
            pltpu.VMEM((IDX_ROWS, 128), jnp.int32),
            pltpu.VMEM((PAIRS, 2 * EMBED), jnp.float32),
            pltpu.SemaphoreType.DMA,
        ],
    )
    def body(pos_hbm, ptable_hbm, out_hbm, pos_v, pidx_v, rows_v, sem):
        num_cores = lax.axis_size("c")
        wid = lax.axis_index("s") * num_cores + lax.axis_index("c")
        base = wid * b_per_w
        lane = lax.iota(jnp.int32, LANES)
        xor1 = lane ^ 1
        evens = (lane * 2) & (LANES - 1)
        lo_half = lane < (LANES // 2)

        def vperm(x, idx):
            return lax.gather(
                x,
                idx[:, None],
                dimension_numbers=lax.GatherDimensionNumbers(
                    offset_dims=(), collapsed_slice_dims=(0,),
                    start_index_map=(0,),
                ),
                slice_sizes=(1,),
                mode=lax.GatherScatterMode.PROMISE_IN_BOUNDS,
            )

        def pair_codes(v):
            r = lax.rem(lax.rem(v, PERIOD) + PERIOD, PERIOD)
            return r * PERIOD + vperm(r, xor1)

        def step(t, carry):
            off = pl.multiple_of(base + t * CHUNK, CHUNK)
            pltpu.sync_copy(pos_hbm.at[pl.ds(off, CHUNK)], pos_v)
            for k in range(PAIRS // LANES):
                ta = pair_codes(pos_v[pl.ds(k * 2 * LANES, LANES)])
                tb = pair_codes(pos_v[pl.ds(k * 2 * LANES + LANES, LANES)])
                ga = vperm(ta, evens)
                gb = vperm(tb, evens)
                pidx_v[k // 8, pl.ds((k % 8) * LANES, LANES)] = jnp.where(
                    lo_half, ga, gb
                )
            cps = [
                pltpu.async_copy(
                    ptable_hbm.at[pidx_v.at[j]],
                    rows_v.at[pl.ds(j * 128, 128)],
                    sem,
                )
                for j in range(IDX_ROWS)
            ]
            for cp in cps:
                cp.wait()
            off2 = pl.multiple_of(base // 2 + t * PAIRS, PAIRS)
            pltpu.sync_copy(rows_v, out_hbm.at[pl.ds(off2, PAIRS)])
            return carry

        lax.fori_loop(0, n_chunks, step, 0)

    return body(pos_flat, ptable)


def kernel(position, embedding):
    info = plsc.get_sparse_core_info()
    num_workers = info.num_cores * info.num_subcores
    ptable = jnp.concatenate(
        [
            jnp.broadcast_to(embedding[:, None, :], (PERIOD, PERIOD, EMBED)),
            jnp.broadcast_to(embedding[None, :, :], (PERIOD, PERIOD, EMBED)),
        ],
        axis=-1,
    ).reshape(PERIOD * PERIOD, 2 * EMBED)
    pos_flat = position.reshape(-1)
    out = _sc_lookup(pos_flat, ptable, num_workers)
    return out.reshape(position.shape + (EMBED,))


# speedup vs baseline: 5.6025x; 1.0138x over previous
import functools

import jax
import jax.numpy as jnp
from jax import lax
from jax.experimental import pallas as pl
from jax.experimental.pallas import tpu as pltpu
from jax.experimental.pallas import tpu_sc as plsc

EMBED = 64
PERIOD = 30
LANES = 16
CHUNK = 512
PAIRS = CHUNK // 2
IDX_ROWS = PAIRS // 128


def _sc_lookup(pos_flat, ptable, num_workers):
    n = pos_flat.shape[0]
    b_per_w = n // num_workers
    n_chunks = b_per_w // CHUNK

    mesh = plsc.VectorSubcoreMesh(core_axis_name="c", subcore_axis_name="s")

    @functools.partial(
        pl.kernel,
        out_type=jax.ShapeDtypeStruct((n // 2, 2 * EMBED), jnp.float32),
        mesh=mesh,
        scratch_types=[
            pltpu.VMEM((2, CHUNK), jnp.int32),
            pltpu.VMEM((2, IDX_ROWS, 128), jnp.int32),
            pltpu.VMEM((2, PAIRS, 2 * EMBED), jnp.float32),
            pltpu.SemaphoreType.DMA,
            pltpu.SemaphoreType.DMA,
        ],
    )
    def body(pos_hbm, ptable_hbm, out_hbm, pos_v, pidx_v, rows_v, sem_g, sem_o):
        num_cores = lax.axis_size("c")
        wid = lax.axis_index("s") * num_cores + lax.axis_index("c")
        base = wid * b_per_w
        base2 = base // 2
        lane = lax.iota(jnp.int32, LANES)
        xor1 = lane ^ 1
        evens = (lane * 2) & (LANES - 1)
        lo_half = lane < (LANES // 2)

        def vperm(x, idx):
            return lax.gather(
                x,
                idx[:, None],
                dimension_numbers=lax.GatherDimensionNumbers(
                    offset_dims=(), collapsed_slice_dims=(0,),
                    start_index_map=(0,),
                ),
                slice_sizes=(1,),
                mode=lax.GatherScatterMode.PROMISE_IN_BOUNDS,
            )

        def pair_codes(v):
            r = lax.rem(lax.rem(v, PERIOD) + PERIOD, PERIOD)
            return r * PERIOD + vperm(r, xor1)

        def load_and_index(u, buf):
            off = pl.multiple_of(base + u * CHUNK, CHUNK)
            pltpu.sync_copy(pos_hbm.at[pl.ds(off, CHUNK)], pos_v.at[buf])
            for k in range(PAIRS // LANES):
                ta = pair_codes(pos_v[buf, pl.ds(k * 2 * LANES, LANES)])
                tb = pair_codes(pos_v[buf, pl.ds(k * 2 * LANES + LANES, LANES)])
                ga = vperm(ta, evens)
                gb = vperm(tb, evens)
                pidx_v[buf, k // 8, pl.ds((k % 8) * LANES, LANES)] = jnp.where(
                    lo_half, ga, gb
                )

        def fire_gathers(buf):
            for j in range(IDX_ROWS):
                pltpu.async_copy(
                    ptable_hbm.at[pidx_v.at[buf, j]],
                    rows_v.at[buf, pl.ds(j * 128, 128)],
                    sem_g,
                )

        def drain_gathers(buf):
            for j in range(IDX_ROWS):
                pltpu.make_async_copy(
                    ptable_hbm.at[pidx_v.at[buf, j]],
                    rows_v.at[buf, pl.ds(j * 128, 128)],
                    sem_g,
                ).wait()

        def drain_scatter():
            pltpu.make_async_copy(
                rows_v.at[0], out_hbm.at[pl.ds(0, PAIRS)], sem_o
            ).wait()

        load_and_index(0, 0)
        fire_gathers(0)

        def step(t, carry):
            buf = lax.rem(t, 2)
            nbuf = lax.rem(t + 1, 2)

            @pl.when(t > 0)
            def _():
                drain_scatter()

            @pl.when(t < n_chunks - 1)
            def _():
                load_and_index(t + 1, nbuf)

            drain_gathers(buf)

            @pl.when(t < n_chunks - 1)
            def _():
                fire_gathers(nbuf)
            off2 = pl.multiple_of(base2 + t * PAIRS, PAIRS)
            pltpu.async_copy(
                rows_v.at[buf], out_hbm.at[pl.ds(off2, PAIRS)], sem_o
            )
            return carry

        lax.fori_loop(0, n_chunks, step, 0)
        drain_scatter()

    return body(pos_flat, ptable)


def kernel(position, embedding):
    info = plsc.get_sparse_core_info()
    num_workers = info.num_cores * info.num_subcores
    ptable = jnp.concatenate(
        [
            jnp.broadcast_to(embedding[:, None, :], (PERIOD, PERIOD, EMBED)),
            jnp.broadcast_to(embedding[None, :, :], (PERIOD, PERIOD, EMBED)),
        ],
        axis=-1,
    ).reshape(PERIOD * PERIOD, 2 * EMBED)
    pos_flat = position.reshape(-1)
    out = _sc_lookup(pos_flat, ptable, num_workers)
    return out.reshape(position.shape + (EMBED,))
